# baseline (speedup 1.0000x reference)
import functools

import jax
import jax.numpy as jnp
from jax import lax
from jax.experimental import pallas as pl
from jax.experimental.pallas import tpu as pltpu
from jax.experimental.pallas import tpu_sc as plsc

N = 10000
NP = 10240
E = 320000
G = 64
EP = 320
H = EP // 2
NL = 5

NC, NS = 2, 16
EW = E // NS
CH = 40
NK = EW // CH
BLK = 10
NB = NK // BLK
RT = NP // NS

HW = EP // 2


def _pack_words(r):
    wa = jnp.concatenate([r[:, 32 * g:32 * g + 16] for g in range(EP // 32)],
                         axis=1)
    wb = jnp.concatenate([r[:, 32 * g + 16:32 * g + 32]
                          for g in range(EP // 32)], axis=1)
    ia = jax.lax.bitcast_convert_type(
        wa.astype(jnp.bfloat16), jnp.int16).astype(jnp.int32) & 0xFFFF
    ib = jax.lax.bitcast_convert_type(
        wb.astype(jnp.bfloat16), jnp.int16).astype(jnp.int32)
    return ia | (ib << 16)


def _encode_body(x_ref, w_ref, b_ref, out_ref, outb_ref):
    r = jnp.dot(x_ref[...], w_ref[...], preferred_element_type=jnp.float32)
    r = r + b_ref[...]
    out_ref[0] = r[:, :H]
    out_ref[1] = r[:, H:]
    ri = _pack_words(r)
    outb_ref[0] = ri[:, :HW // 2]
    outb_ref[1] = ri[:, HW // 2:]


def _encode(x, node_Wp, node_bp):
    bm = 1024
    return pl.pallas_call(
        _encode_body,
        grid=(NP // bm,),
        in_specs=[
            pl.BlockSpec((bm, 128), lambda i: (i, 0)),
            pl.BlockSpec((128, EP), lambda i: (0, 0)),
            pl.BlockSpec((1, EP), lambda i: (0, 0)),
        ],
        out_specs=[
            pl.BlockSpec((NC, bm, H), lambda i: (0, i, 0)),
            pl.BlockSpec((NC, bm, HW // 2), lambda i: (0, i, 0)),
        ],
        out_shape=[
            jax.ShapeDtypeStruct((NC, NP, H), jnp.float32),
            jax.ShapeDtypeStruct((NC, NP, HW // 2), jnp.int32),
        ],
    )(x, node_Wp, node_bp)


def _edge_mlp_body(a_ref, w_ref, b_ref, out_ref):
    a16 = a_ref[...].astype(jnp.bfloat16)
    w16 = w_ref[...].astype(jnp.bfloat16)
    r = jnp.dot(a16, w16, preferred_element_type=jnp.float32)
    r16 = jnp.maximum(r + b_ref[...], 0.0).astype(jnp.bfloat16)
    out_ref[0] = r16[:, :H]
    out_ref[1] = r16[:, H:]


def _edge_mlp(edge_attr, Wl, bl):
    bm = 2000
    return pl.pallas_call(
        _edge_mlp_body,
        grid=(E // bm,),
        in_specs=[
            pl.BlockSpec((bm, 16), lambda i: (i, 0)),
            pl.BlockSpec((16, EP), lambda i: (0, 0)),
            pl.BlockSpec((1, EP), lambda i: (0, 0)),
        ],
        out_specs=pl.BlockSpec((NC, bm, H), lambda i: (0, i, 0)),
        out_shape=jax.ShapeDtypeStruct((NC, E, H), jnp.bfloat16),
    )(edge_attr, Wl, bl)


def _mlp_body(last, h_ref, agg_ref, w1_ref, b1_ref, w2_ref, b2_ref, eps_ref,
              out_ref, outb_ref):
    hb = jnp.concatenate([h_ref[0], h_ref[1]], axis=1)
    ab = jnp.concatenate([agg_ref[0], agg_ref[1]], axis=1)
    h2 = (1.0 + eps_ref[0, 0]) * hb + ab
    t = jnp.maximum(
        jnp.dot(h2, w1_ref[...], preferred_element_type=jnp.float32)
        + b1_ref[...], 0.0)
    o = jnp.dot(t, w2_ref[...], preferred_element_type=jnp.float32) + b2_ref[...]
    if not last:
        o = jnp.maximum(o, 0.0)
    out_ref[0] = o[:, :H]
    out_ref[1] = o[:, H:]
    oi = _pack_words(o)
    outb_ref[0] = oi[:, :HW // 2]
    outb_ref[1] = oi[:, HW // 2:]


def _gin_mlp(h, agg, W1l, b1l, W2l, b2l, epsl, last):
    bm = 1024
    return pl.pallas_call(
        functools.partial(_mlp_body, last),
        grid=(NP // bm,),
        in_specs=[
            pl.BlockSpec((NC, bm, H), lambda i: (0, i, 0)),
            pl.BlockSpec((NC, bm, H), lambda i: (0, i, 0)),
            pl.BlockSpec((EP, 640), lambda i: (0, 0)),
            pl.BlockSpec((1, 640), lambda i: (0, 0)),
            pl.BlockSpec((640, EP), lambda i: (0, 0)),
            pl.BlockSpec((1, EP), lambda i: (0, 0)),
            pl.BlockSpec((1, 1), lambda i: (0, 0), memory_space=pltpu.SMEM),
        ],
        out_specs=[
            pl.BlockSpec((NC, bm, H), lambda i: (0, i, 0)),
            pl.BlockSpec((NC, bm, HW // 2), lambda i: (0, i, 0)),
        ],
        out_shape=[
            jax.ShapeDtypeStruct((NC, NP, H), jnp.float32),
            jax.ShapeDtypeStruct((NC, NP, HW // 2), jnp.int32),
        ],
    )(h, agg, W1l, b1l, W2l, b2l, epsl)


def _pool_body(h_ref, batch_ref, pw_ref, pb_ref, pred_ref, norm_ref,
               acc_ref, cnt_ref):
    i = pl.program_id(0)

    @pl.when(i == 0)
    def _init():
        acc_ref[...] = jnp.zeros_like(acc_ref)
        cnt_ref[...] = jnp.zeros_like(cnt_ref)

    hb = jnp.concatenate([h_ref[0], h_ref[1]], axis=1)
    bm = hb.shape[0]
    gids = jax.lax.broadcasted_iota(jnp.int32, (bm, G), 1)
    onehot = (batch_ref[0, 0][:, None] == gids).astype(jnp.float32)
    acc_ref[...] += jax.lax.dot_general(
        onehot, hb, (((0,), (0,)), ((), ())),
        preferred_element_type=jnp.float32)
    cnt_ref[...] += jax.lax.dot_general(
        onehot, jnp.ones((bm, 128), jnp.float32), (((0,), (0,)), ((), ())),
        preferred_element_type=jnp.float32)

    @pl.when(i == pl.num_programs(0) - 1)
    def _done():
        cnt = jnp.maximum(cnt_ref[:, 0:1], 1.0)
        hg = acc_ref[...] / cnt
        logit = jnp.dot(hg, pw_ref[...], preferred_element_type=jnp.float32)
        p = logit[:, 0] + pb_ref[0, 0]
        pred_ref[0] = p
        norm_ref[0] = 2.0 * ((p - (-10.0)) / 6.0) - 1.0


def _pool_head(h, batch2d, pred_Wp, pred_b2d):
    bm = 1024
    return pl.pallas_call(
        _pool_body,
        grid=(NP // bm,),
        in_specs=[
            pl.BlockSpec((NC, bm, H), lambda i: (0, i, 0)),
            pl.BlockSpec((1, 1, bm), lambda i: (i, 0, 0)),
            pl.BlockSpec((EP, 128), lambda i: (0, 0)),
            pl.BlockSpec((1, 1), lambda i: (0, 0), memory_space=pltpu.SMEM),
        ],
        out_specs=[
            pl.BlockSpec((1, G), lambda i: (0, 0)),
            pl.BlockSpec((1, G), lambda i: (0, 0)),
        ],
        out_shape=[
            jax.ShapeDtypeStruct((1, G), jnp.float32),
            jax.ShapeDtypeStruct((1, G), jnp.float32),
        ],
        scratch_shapes=[
            pltpu.VMEM((G, EP), jnp.float32),
            pltpu.VMEM((G, 128), jnp.float32),
        ],
    )(h, batch2d, pred_Wp, pred_b2d)


def _sc_layer_body(h_hbm, e_hbm, src_hbm, dst_hbm, out_hbm,
                   acc, src_blk, dst_blk, rows_v, e_v, msg_v,
                   idx_sem, e_sem, gat_sem, sct_sem):
    c = lax.axis_index("c")
    s = lax.axis_index("s")

    def _zrow(i, carry):
        for v in range(H // 16):
            msg_v[0, i, pl.ds(v * 16, 16)] = jnp.zeros((16,), jnp.float32)
        return carry
    lax.fori_loop(0, CH, _zrow, None)
    for j in range(RT // CH):
        pltpu.sync_copy(msg_v.at[0], acc.at[pl.ds(s * RT + j * CH, CH)])

    plsc.subcore_barrier()

    def _issue_blk(bnum):
        slot = lax.rem(bnum, 3)
        pltpu.async_copy(src_hbm.at[s, pl.ds(bnum * BLK, BLK)],
                         src_blk.at[slot], idx_sem.at[slot])
        pltpu.async_copy(dst_hbm.at[s, pl.ds(bnum * BLK, BLK)],
                         dst_blk.at[slot], idx_sem.at[slot])

    def _wait_blk(bnum):
        slot = lax.rem(bnum, 3)
        pltpu.make_async_copy(src_hbm.at[s, pl.ds(0, BLK)],
                              src_blk.at[slot], idx_sem.at[slot]).wait()
        pltpu.make_async_copy(dst_hbm.at[s, pl.ds(0, BLK)],
                              dst_blk.at[slot], idx_sem.at[slot]).wait()

    def _issue_e(k):
        b = lax.rem(k, 2)
        pltpu.async_copy(e_hbm.at[c, pl.ds(s * EW + k * CH, CH)],
                         e_v.at[b], e_sem.at[b])

    def _issue_gather(k):
        b = lax.rem(k, 2)
        slot = lax.rem(k // BLK, 3)
        g = lax.rem(k, BLK)
        pltpu.async_copy(h_hbm.at[c].at[src_blk.at[slot, g]],
                         rows_v.at[b], gat_sem.at[b])

    def _wait_scatter(b):
        pltpu.make_async_copy(msg_v.at[b], acc.at[dst_blk.at[0, 0]],
                              sct_sem.at[b]).wait()

    _issue_blk(0)
    _issue_blk(1)
    _wait_blk(0)
    _issue_e(0)
    _issue_gather(0)

    def _chunk(k, carry):
        b = lax.rem(k, 2)
        slot = lax.rem(k // BLK, 3)
        g = lax.rem(k, BLK)

        pltpu.make_async_copy(h_hbm.at[c].at[src_blk.at[slot, g]],
                              rows_v.at[b], gat_sem.at[b]).wait()
        pltpu.make_async_copy(e_hbm.at[c, pl.ds(0, CH)], e_v.at[b],
                              e_sem.at[b]).wait()

        @pl.when(k >= 2)
        def _():
            _wait_scatter(b)

        @pl.when(lax.rem(k + 1, BLK) == 0)
        def _():
            bnext = (k + 1) // BLK

            @pl.when(bnext < NB)
            def _():
                _wait_blk(bnext)

            @pl.when(bnext + 1 < NB)
            def _():
                _issue_blk(bnext + 1)

        @pl.when(k + 1 < NK)
        def _():
            _issue_gather(k + 1)
            _issue_e(k + 1)

        @plsc.parallel_loop(0, CH, 1, unroll=4)
        def _edge(j):
            for v in range(H // 32):
                sl = pl.ds(v * 16, 16)
                rw = rows_v[b, j, sl]
                ew = plsc.bitcast(e_v[b, j, pl.ds(v * 32, 32)], jnp.int32)
                ra = jax.lax.bitcast_convert_type(rw << 16, jnp.float32)
                rb = jax.lax.bitcast_convert_type(rw & -65536, jnp.float32)
                ea = jax.lax.bitcast_convert_type(ew << 16, jnp.float32)
                eb = jax.lax.bitcast_convert_type(ew & -65536, jnp.float32)
                msg_v[b, j, pl.ds(v * 32, 16)] = jnp.maximum(ra + ea, 0.0)
                msg_v[b, j, pl.ds(v * 32 + 16, 16)] = jnp.maximum(rb + eb, 0.0)

        pltpu.async_copy(msg_v.at[b], acc.at[dst_blk.at[slot, g]],
                         sct_sem.at[b], add=True)
        return carry
    lax.fori_loop(0, NK, _chunk, None)

    _wait_scatter(lax.rem(NK - 2, 2))
    _wait_scatter(lax.rem(NK - 1, 2))

    plsc.subcore_barrier()

    pltpu.sync_copy(acc.at[pl.ds(s * RT, RT)],
                    out_hbm.at[c].at[pl.ds(s * RT, RT)])


def _sc_layer(h_bf, e_bf, src_t, dst_t):
    mesh = plsc.VectorSubcoreMesh(core_axis_name="c", subcore_axis_name="s",
                                  num_cores=NC, num_subcores=NS)
    f = pl.kernel(
        _sc_layer_body,
        out_type=jax.ShapeDtypeStruct((NC, NP, H), jnp.float32),
        mesh=mesh,
        compiler_params=pltpu.CompilerParams(use_tc_tiling_on_sc=False, needs_layout_passes=False),
        scratch_types=[
            pltpu.VMEM_SHARED((NP, H), jnp.float32),
            pltpu.VMEM((3, BLK, CH), jnp.int32),
            pltpu.VMEM((3, BLK, CH), jnp.int32),
            pltpu.VMEM((2, CH, HW // 2), jnp.int32),
            pltpu.VMEM((2, CH, H), jnp.bfloat16),
            pltpu.VMEM((2, CH, H), jnp.float32),
            pltpu.SemaphoreType.DMA((3,)),
            pltpu.SemaphoreType.DMA((2,)),
            pltpu.SemaphoreType.DMA((2,)),
            pltpu.SemaphoreType.DMA((2,)),
        ],
    )
    return f(h_bf, e_bf, src_t, dst_t)


def kernel(x, edge_index, edge_attr, batch, node_W, node_b, edge_W, edge_b,
           W1, b1, W2, b2, eps, pred_W, pred_b):
    f32 = jnp.float32
    node_Wp = jnp.zeros((128, EP), f32).at[:, :300].set(node_W)
    node_bp = jnp.zeros((1, EP), f32).at[0, :300].set(node_b)
    edge_Wp = jnp.zeros((NL, 16, EP), f32).at[:, :, :300].set(edge_W)
    edge_bp = jnp.zeros((NL, 1, EP), f32).at[:, 0, :300].set(edge_b)
    tau = jnp.array([32 * g + 16 * p + i
                     for g in range(EP // 32) for i in range(16)
                     for p in range(2)], dtype=jnp.int32)
    edge_Wp = edge_Wp[:, :, tau]
    edge_bp = edge_bp[:, :, tau]
    W1p = jnp.zeros((NL, EP, 640), f32).at[:, :300, :600].set(W1)
    b1p = jnp.zeros((NL, 1, 640), f32).at[:, 0, :600].set(b1)
    W2p = jnp.zeros((NL, 640, EP), f32).at[:, :600, :300].set(W2)
    b2p = jnp.zeros((NL, 1, EP), f32).at[:, 0, :300].set(b2)
    pred_Wp = jnp.zeros((EP, 128), f32).at[:300, 0].set(pred_W[:, 0])
    pred_b2d = pred_b.reshape(1, 1)
    eps2d = eps.reshape(NL, 1, 1)

    src_t = edge_index[0].reshape(NS, NK, CH)
    dst_t = edge_index[1].reshape(NS, NK, CH)
    x_p = jnp.zeros((NP, 128), f32).at[:N].set(x)
    batch_p = jnp.full((NP,), -1, jnp.int32).at[:N].set(batch)
    batch2d = batch_p.reshape(NP // 1024, 1, 1024)

    h, h_bf = _encode(x_p, node_Wp, node_bp)
    e_bfs = [_edge_mlp(edge_attr, edge_Wp[l], edge_bp[l]) for l in range(NL)]
    for l in range(NL):
        agg = _sc_layer(h_bf, e_bfs[l], src_t, dst_t)
        h, h_bf = _gin_mlp(h, agg, W1p[l], b1p[l], W2p[l], b2p[l], eps2d[l],
                           last=(l == NL - 1))

    pred2d, norm2d = _pool_head(h, batch2d, pred_Wp, pred_b2d)
    return (pred2d[0], norm2d[0])


# speedup vs baseline: 1.3715x; 1.3715x over previous
import functools

import jax
import jax.numpy as jnp
from jax import lax
from jax.experimental import pallas as pl
from jax.experimental.pallas import tpu as pltpu
from jax.experimental.pallas import tpu_sc as plsc

N = 10000
NP = 10240
E = 320000
G = 64
EP = 320
H = EP // 2
NL = 5

NC, NS = 2, 16
EW = E // NS
CH = 40
NK = EW // CH
BLK = 10
NB = NK // BLK
RT = NP // NS

HW = EP // 2


def _pack_words(r):
    wa = jnp.concatenate([r[:, 32 * g:32 * g + 16] for g in range(EP // 32)],
                         axis=1)
    wb = jnp.concatenate([r[:, 32 * g + 16:32 * g + 32]
                          for g in range(EP // 32)], axis=1)
    ia = jax.lax.bitcast_convert_type(
        wa.astype(jnp.bfloat16), jnp.int16).astype(jnp.int32) & 0xFFFF
    ib = jax.lax.bitcast_convert_type(
        wb.astype(jnp.bfloat16), jnp.int16).astype(jnp.int32)
    return ia | (ib << 16)


def _encode_body(x_ref, w_ref, b_ref, out_ref, outb_ref):
    r = jnp.dot(x_ref[...], w_ref[...], preferred_element_type=jnp.float32)
    r = r + b_ref[...]
    out_ref[0] = r[:, :H]
    out_ref[1] = r[:, H:]
    ri = _pack_words(r)
    outb_ref[0] = ri[:, :HW // 2]
    outb_ref[1] = ri[:, HW // 2:]


def _encode(x, node_Wp, node_bp):
    bm = 1024
    return pl.pallas_call(
        _encode_body,
        grid=(NP // bm,),
        in_specs=[
            pl.BlockSpec((bm, 128), lambda i: (i, 0)),
            pl.BlockSpec((128, EP), lambda i: (0, 0)),
            pl.BlockSpec((1, EP), lambda i: (0, 0)),
        ],
        out_specs=[
            pl.BlockSpec((NC, bm, H), lambda i: (0, i, 0)),
            pl.BlockSpec((NC, bm, HW // 2), lambda i: (0, i, 0)),
        ],
        out_shape=[
            jax.ShapeDtypeStruct((NC, NP, H), jnp.float32),
            jax.ShapeDtypeStruct((NC, NP, HW // 2), jnp.int32),
        ],
    )(x, node_Wp, node_bp)


def _edge_mlp_body(a_ref, w_ref, b_ref, out_ref):
    a16 = a_ref[...].astype(jnp.bfloat16)
    w16 = w_ref[...].astype(jnp.bfloat16)
    r = jnp.dot(a16, w16, preferred_element_type=jnp.float32)
    r = jnp.maximum(r + b_ref[...], 0.0)
    ri = _pack_words(r)
    out_ref[0] = ri[:, :HW // 2]
    out_ref[1] = ri[:, HW // 2:]


def _edge_mlp(edge_attr, Wl, bl):
    bm = 2000
    return pl.pallas_call(
        _edge_mlp_body,
        grid=(E // bm,),
        in_specs=[
            pl.BlockSpec((bm, 16), lambda i: (i, 0)),
            pl.BlockSpec((16, EP), lambda i: (0, 0)),
            pl.BlockSpec((1, EP), lambda i: (0, 0)),
        ],
        out_specs=pl.BlockSpec((NC, bm, HW // 2), lambda i: (0, i, 0)),
        out_shape=jax.ShapeDtypeStruct((NC, E, HW // 2), jnp.int32),
    )(edge_attr, Wl, bl)


def _mlp_body(last, h_ref, agg_ref, w1_ref, b1_ref, w2_ref, b2_ref, eps_ref,
              out_ref, outb_ref):
    hb = jnp.concatenate([h_ref[0], h_ref[1]], axis=1)
    ab = jnp.concatenate([agg_ref[0], agg_ref[1]], axis=1)
    h2 = (1.0 + eps_ref[0, 0]) * hb + ab
    t = jnp.maximum(
        jnp.dot(h2, w1_ref[...], preferred_element_type=jnp.float32)
        + b1_ref[...], 0.0)
    o = jnp.dot(t, w2_ref[...], preferred_element_type=jnp.float32) + b2_ref[...]
    if not last:
        o = jnp.maximum(o, 0.0)
    out_ref[0] = o[:, :H]
    out_ref[1] = o[:, H:]
    oi = _pack_words(o)
    outb_ref[0] = oi[:, :HW // 2]
    outb_ref[1] = oi[:, HW // 2:]


def _gin_mlp(h, agg, W1l, b1l, W2l, b2l, epsl, last):
    bm = 1024
    return pl.pallas_call(
        functools.partial(_mlp_body, last),
        grid=(NP // bm,),
        in_specs=[
            pl.BlockSpec((NC, bm, H), lambda i: (0, i, 0)),
            pl.BlockSpec((NC, bm, H), lambda i: (0, i, 0)),
            pl.BlockSpec((EP, 640), lambda i: (0, 0)),
            pl.BlockSpec((1, 640), lambda i: (0, 0)),
            pl.BlockSpec((640, EP), lambda i: (0, 0)),
            pl.BlockSpec((1, EP), lambda i: (0, 0)),
            pl.BlockSpec((1, 1), lambda i: (0, 0), memory_space=pltpu.SMEM),
        ],
        out_specs=[
            pl.BlockSpec((NC, bm, H), lambda i: (0, i, 0)),
            pl.BlockSpec((NC, bm, HW // 2), lambda i: (0, i, 0)),
        ],
        out_shape=[
            jax.ShapeDtypeStruct((NC, NP, H), jnp.float32),
            jax.ShapeDtypeStruct((NC, NP, HW // 2), jnp.int32),
        ],
    )(h, agg, W1l, b1l, W2l, b2l, epsl)


def _pool_body(h_ref, batch_ref, pw_ref, pb_ref, pred_ref, norm_ref,
               acc_ref, cnt_ref):
    i = pl.program_id(0)

    @pl.when(i == 0)
    def _init():
        acc_ref[...] = jnp.zeros_like(acc_ref)
        cnt_ref[...] = jnp.zeros_like(cnt_ref)

    hb = jnp.concatenate([h_ref[0], h_ref[1]], axis=1)
    bm = hb.shape[0]
    gids = jax.lax.broadcasted_iota(jnp.int32, (bm, G), 1)
    onehot = (batch_ref[0, 0][:, None] == gids).astype(jnp.float32)
    acc_ref[...] += jax.lax.dot_general(
        onehot, hb, (((0,), (0,)), ((), ())),
        preferred_element_type=jnp.float32)
    cnt_ref[...] += jax.lax.dot_general(
        onehot, jnp.ones((bm, 128), jnp.float32), (((0,), (0,)), ((), ())),
        preferred_element_type=jnp.float32)

    @pl.when(i == pl.num_programs(0) - 1)
    def _done():
        cnt = jnp.maximum(cnt_ref[:, 0:1], 1.0)
        hg = acc_ref[...] / cnt
        logit = jnp.dot(hg, pw_ref[...], preferred_element_type=jnp.float32)
        p = logit[:, 0] + pb_ref[0, 0]
        pred_ref[0] = p
        norm_ref[0] = 2.0 * ((p - (-10.0)) / 6.0) - 1.0


def _pool_head(h, batch2d, pred_Wp, pred_b2d):
    bm = 1024
    return pl.pallas_call(
        _pool_body,
        grid=(NP // bm,),
        in_specs=[
            pl.BlockSpec((NC, bm, H), lambda i: (0, i, 0)),
            pl.BlockSpec((1, 1, bm), lambda i: (i, 0, 0)),
            pl.BlockSpec((EP, 128), lambda i: (0, 0)),
            pl.BlockSpec((1, 1), lambda i: (0, 0), memory_space=pltpu.SMEM),
        ],
        out_specs=[
            pl.BlockSpec((1, G), lambda i: (0, 0)),
            pl.BlockSpec((1, G), lambda i: (0, 0)),
        ],
        out_shape=[
            jax.ShapeDtypeStruct((1, G), jnp.float32),
            jax.ShapeDtypeStruct((1, G), jnp.float32),
        ],
        scratch_shapes=[
            pltpu.VMEM((G, EP), jnp.float32),
            pltpu.VMEM((G, 128), jnp.float32),
        ],
    )(h, batch2d, pred_Wp, pred_b2d)


def _sc_layer_body(h_hbm, e_hbm, src_hbm, dst_hbm, out_hbm,
                   acc, src_blk, dst_blk, rows_v, e_v, msg_v,
                   idx_sem, e_sem, gat_sem, sct_sem):
    c = lax.axis_index("c")
    s = lax.axis_index("s")

    def _zrow(i, carry):
        for v in range(H // 16):
            msg_v[0, i, pl.ds(v * 16, 16)] = jnp.zeros((16,), jnp.float32)
        return carry
    lax.fori_loop(0, CH, _zrow, None)
    for j in range(RT // CH):
        pltpu.sync_copy(msg_v.at[0], acc.at[pl.ds(s * RT + j * CH, CH)])

    plsc.subcore_barrier()

    def _issue_blk(bnum):
        slot = lax.rem(bnum, 3)
        pltpu.async_copy(src_hbm.at[s, pl.ds(bnum * BLK, BLK)],
                         src_blk.at[slot], idx_sem.at[slot])
        pltpu.async_copy(dst_hbm.at[s, pl.ds(bnum * BLK, BLK)],
                         dst_blk.at[slot], idx_sem.at[slot])

    def _wait_blk(bnum):
        slot = lax.rem(bnum, 3)
        pltpu.make_async_copy(src_hbm.at[s, pl.ds(0, BLK)],
                              src_blk.at[slot], idx_sem.at[slot]).wait()
        pltpu.make_async_copy(dst_hbm.at[s, pl.ds(0, BLK)],
                              dst_blk.at[slot], idx_sem.at[slot]).wait()

    def _issue_e(k):
        b = lax.rem(k, 2)
        pltpu.async_copy(e_hbm.at[c, pl.ds(s * EW + k * CH, CH)],
                         e_v.at[b], e_sem.at[b])

    def _issue_gather(k):
        b = lax.rem(k, 2)
        slot = lax.rem(k // BLK, 3)
        g = lax.rem(k, BLK)
        pltpu.async_copy(h_hbm.at[c].at[src_blk.at[slot, g]],
                         rows_v.at[b], gat_sem.at[b])

    def _wait_scatter(b):
        pltpu.make_async_copy(msg_v.at[b], acc.at[dst_blk.at[0, 0]],
                              sct_sem.at[b]).wait()

    _issue_blk(0)
    _issue_blk(1)
    _wait_blk(0)
    _issue_e(0)
    _issue_gather(0)

    def _chunk(k, carry):
        b = lax.rem(k, 2)
        slot = lax.rem(k // BLK, 3)
        g = lax.rem(k, BLK)

        pltpu.make_async_copy(h_hbm.at[c].at[src_blk.at[slot, g]],
                              rows_v.at[b], gat_sem.at[b]).wait()
        pltpu.make_async_copy(e_hbm.at[c, pl.ds(0, CH)], e_v.at[b],
                              e_sem.at[b]).wait()

        @pl.when(k >= 2)
        def _():
            _wait_scatter(b)

        @pl.when(lax.rem(k + 1, BLK) == 0)
        def _():
            bnext = (k + 1) // BLK

            @pl.when(bnext < NB)
            def _():
                _wait_blk(bnext)

            @pl.when(bnext + 1 < NB)
            def _():
                _issue_blk(bnext + 1)

        @pl.when(k + 1 < NK)
        def _():
            _issue_gather(k + 1)
            _issue_e(k + 1)

        @plsc.parallel_loop(0, CH, 1, unroll=4)
        def _edge(j):
            for v in range(H // 32):
                sl = pl.ds(v * 16, 16)
                rw = rows_v[b, j, sl]
                ew = e_v[b, j, sl]
                ra = jax.lax.bitcast_convert_type(rw << 16, jnp.float32)
                rb = jax.lax.bitcast_convert_type(rw & -65536, jnp.float32)
                ea = jax.lax.bitcast_convert_type(ew << 16, jnp.float32)
                eb = jax.lax.bitcast_convert_type(ew & -65536, jnp.float32)
                msg_v[b, j, pl.ds(v * 32, 16)] = jnp.maximum(ra + ea, 0.0)
                msg_v[b, j, pl.ds(v * 32 + 16, 16)] = jnp.maximum(rb + eb, 0.0)

        pltpu.async_copy(msg_v.at[b], acc.at[dst_blk.at[slot, g]],
                         sct_sem.at[b], add=True)
        return carry
    lax.fori_loop(0, NK, _chunk, None)

    _wait_scatter(lax.rem(NK - 2, 2))
    _wait_scatter(lax.rem(NK - 1, 2))

    plsc.subcore_barrier()

    pltpu.sync_copy(acc.at[pl.ds(s * RT, RT)],
                    out_hbm.at[c].at[pl.ds(s * RT, RT)])


def _sc_layer(h_bf, e_bf, src_t, dst_t):
    mesh = plsc.VectorSubcoreMesh(core_axis_name="c", subcore_axis_name="s",
                                  num_cores=NC, num_subcores=NS)
    f = pl.kernel(
        _sc_layer_body,
        out_type=jax.ShapeDtypeStruct((NC, NP, H), jnp.float32),
        mesh=mesh,
        compiler_params=pltpu.CompilerParams(use_tc_tiling_on_sc=False, needs_layout_passes=False),
        scratch_types=[
            pltpu.VMEM_SHARED((NP, H), jnp.float32),
            pltpu.VMEM((3, BLK, CH), jnp.int32),
            pltpu.VMEM((3, BLK, CH), jnp.int32),
            pltpu.VMEM((2, CH, HW // 2), jnp.int32),
            pltpu.VMEM((2, CH, HW // 2), jnp.int32),
            pltpu.VMEM((2, CH, H), jnp.float32),
            pltpu.SemaphoreType.DMA((3,)),
            pltpu.SemaphoreType.DMA((2,)),
            pltpu.SemaphoreType.DMA((2,)),
            pltpu.SemaphoreType.DMA((2,)),
        ],
    )
    return f(h_bf, e_bf, src_t, dst_t)


def kernel(x, edge_index, edge_attr, batch, node_W, node_b, edge_W, edge_b,
           W1, b1, W2, b2, eps, pred_W, pred_b):
    f32 = jnp.float32
    node_Wp = jnp.zeros((128, EP), f32).at[:, :300].set(node_W)
    node_bp = jnp.zeros((1, EP), f32).at[0, :300].set(node_b)
    edge_Wp = jnp.zeros((NL, 16, EP), f32).at[:, :, :300].set(edge_W)
    edge_bp = jnp.zeros((NL, 1, EP), f32).at[:, 0, :300].set(edge_b)
    W1p = jnp.zeros((NL, EP, 640), f32).at[:, :300, :600].set(W1)
    b1p = jnp.zeros((NL, 1, 640), f32).at[:, 0, :600].set(b1)
    W2p = jnp.zeros((NL, 640, EP), f32).at[:, :600, :300].set(W2)
    b2p = jnp.zeros((NL, 1, EP), f32).at[:, 0, :300].set(b2)
    pred_Wp = jnp.zeros((EP, 128), f32).at[:300, 0].set(pred_W[:, 0])
    pred_b2d = pred_b.reshape(1, 1)
    eps2d = eps.reshape(NL, 1, 1)

    src_t = edge_index[0].reshape(NS, NK, CH)
    dst_t = edge_index[1].reshape(NS, NK, CH)
    x_p = jnp.zeros((NP, 128), f32).at[:N].set(x)
    batch_p = jnp.full((NP,), -1, jnp.int32).at[:N].set(batch)
    batch2d = batch_p.reshape(NP // 1024, 1, 1024)

    h, h_bf = _encode(x_p, node_Wp, node_bp)
    e_bfs = [_edge_mlp(edge_attr, edge_Wp[l], edge_bp[l]) for l in range(NL)]
    for l in range(NL):
        agg = _sc_layer(h_bf, e_bfs[l], src_t, dst_t)
        h, h_bf = _gin_mlp(h, agg, W1p[l], b1p[l], W2p[l], b2p[l], eps2d[l],
                           last=(l == NL - 1))

    pred2d, norm2d = _pool_head(h, batch2d, pred_Wp, pred_b2d)
    return (pred2d[0], norm2d[0])


# speedup vs baseline: 1.3720x; 1.0004x over previous
import functools

import jax
import jax.numpy as jnp
from jax import lax
from jax.experimental import pallas as pl
from jax.experimental.pallas import tpu as pltpu
from jax.experimental.pallas import tpu_sc as plsc

N = 10000
NP = 10240
E = 320000
G = 64
EP = 320
H = EP // 2
NL = 5

NC, NS = 2, 16
EW = E // NS
CH = 40
NK = EW // CH
BLK = 10
NB = NK // BLK
RT = NP // NS

HW = EP // 2


def _pack_words(r):
    wa = jnp.concatenate([r[:, 32 * g:32 * g + 16] for g in range(EP // 32)],
                         axis=1)
    wb = jnp.concatenate([r[:, 32 * g + 16:32 * g + 32]
                          for g in range(EP // 32)], axis=1)
    ia = jax.lax.bitcast_convert_type(
        wa.astype(jnp.bfloat16), jnp.int16).astype(jnp.int32) & 0xFFFF
    ib = jax.lax.bitcast_convert_type(
        wb.astype(jnp.bfloat16), jnp.int16).astype(jnp.int32)
    return ia | (ib << 16)


def _encode_body(x_ref, w_ref, b_ref, out_ref, outb_ref):
    r = jnp.dot(x_ref[...], w_ref[...], preferred_element_type=jnp.float32)
    r = r + b_ref[...]
    out_ref[0] = r[:, :H]
    out_ref[1] = r[:, H:]
    ri = _pack_words(r)
    outb_ref[0] = ri[:, :HW // 2]
    outb_ref[1] = ri[:, HW // 2:]


def _encode(x, node_Wp, node_bp):
    bm = 1024
    return pl.pallas_call(
        _encode_body,
        grid=(NP // bm,),
        in_specs=[
            pl.BlockSpec((bm, 128), lambda i: (i, 0)),
            pl.BlockSpec((128, EP), lambda i: (0, 0)),
            pl.BlockSpec((1, EP), lambda i: (0, 0)),
        ],
        out_specs=[
            pl.BlockSpec((NC, bm, H), lambda i: (0, i, 0)),
            pl.BlockSpec((NC, bm, HW // 2), lambda i: (0, i, 0)),
        ],
        out_shape=[
            jax.ShapeDtypeStruct((NC, NP, H), jnp.float32),
            jax.ShapeDtypeStruct((NC, NP, HW // 2), jnp.int32),
        ],
    )(x, node_Wp, node_bp)


def _edge_mlp_body(a_ref, w_ref, b_ref, out_ref):
    a16 = a_ref[...].astype(jnp.bfloat16)
    w16 = w_ref[...].astype(jnp.bfloat16)
    r = jnp.dot(a16, w16, preferred_element_type=jnp.float32)
    r = jnp.maximum(r + b_ref[...], 0.0)
    ri = _pack_words(r)
    out_ref[0] = ri[:, :HW // 2]
    out_ref[1] = ri[:, HW // 2:]


def _edge_mlp(edge_attr, Wl, bl):
    bm = 2000
    return pl.pallas_call(
        _edge_mlp_body,
        grid=(E // bm,),
        in_specs=[
            pl.BlockSpec((bm, 16), lambda i: (i, 0)),
            pl.BlockSpec((16, EP), lambda i: (0, 0)),
            pl.BlockSpec((1, EP), lambda i: (0, 0)),
        ],
        out_specs=pl.BlockSpec((NC, bm, HW // 2), lambda i: (0, i, 0)),
        out_shape=jax.ShapeDtypeStruct((NC, E, HW // 2), jnp.int32),
    )(edge_attr, Wl, bl)


def _mlp_body(last, h_ref, agg_ref, w1_ref, b1_ref, w2_ref, b2_ref, eps_ref,
              out_ref, outb_ref):
    hb = jnp.concatenate([h_ref[0], h_ref[1]], axis=1)
    ab = jnp.concatenate([agg_ref[0], agg_ref[1]], axis=1)
    h2 = (1.0 + eps_ref[0, 0]) * hb + ab
    t = jnp.maximum(
        jnp.dot(h2, w1_ref[...], preferred_element_type=jnp.float32)
        + b1_ref[...], 0.0)
    o = jnp.dot(t, w2_ref[...], preferred_element_type=jnp.float32) + b2_ref[...]
    if not last:
        o = jnp.maximum(o, 0.0)
    out_ref[0] = o[:, :H]
    out_ref[1] = o[:, H:]
    oi = _pack_words(o)
    outb_ref[0] = oi[:, :HW // 2]
    outb_ref[1] = oi[:, HW // 2:]


def _gin_mlp(h, agg, W1l, b1l, W2l, b2l, epsl, last):
    bm = 1024
    return pl.pallas_call(
        functools.partial(_mlp_body, last),
        grid=(NP // bm,),
        in_specs=[
            pl.BlockSpec((NC, bm, H), lambda i: (0, i, 0)),
            pl.BlockSpec((NC, bm, H), lambda i: (0, i, 0)),
            pl.BlockSpec((EP, 640), lambda i: (0, 0)),
            pl.BlockSpec((1, 640), lambda i: (0, 0)),
            pl.BlockSpec((640, EP), lambda i: (0, 0)),
            pl.BlockSpec((1, EP), lambda i: (0, 0)),
            pl.BlockSpec((1, 1), lambda i: (0, 0), memory_space=pltpu.SMEM),
        ],
        out_specs=[
            pl.BlockSpec((NC, bm, H), lambda i: (0, i, 0)),
            pl.BlockSpec((NC, bm, HW // 2), lambda i: (0, i, 0)),
        ],
        out_shape=[
            jax.ShapeDtypeStruct((NC, NP, H), jnp.float32),
            jax.ShapeDtypeStruct((NC, NP, HW // 2), jnp.int32),
        ],
    )(h, agg, W1l, b1l, W2l, b2l, epsl)


def _pool_body(h_ref, batch_ref, pw_ref, pb_ref, pred_ref, norm_ref,
               acc_ref, cnt_ref):
    i = pl.program_id(0)

    @pl.when(i == 0)
    def _init():
        acc_ref[...] = jnp.zeros_like(acc_ref)
        cnt_ref[...] = jnp.zeros_like(cnt_ref)

    hb = jnp.concatenate([h_ref[0], h_ref[1]], axis=1)
    bm = hb.shape[0]
    gids = jax.lax.broadcasted_iota(jnp.int32, (bm, G), 1)
    onehot = (batch_ref[0, 0][:, None] == gids).astype(jnp.float32)
    acc_ref[...] += jax.lax.dot_general(
        onehot, hb, (((0,), (0,)), ((), ())),
        preferred_element_type=jnp.float32)
    cnt_ref[...] += jax.lax.dot_general(
        onehot, jnp.ones((bm, 128), jnp.float32), (((0,), (0,)), ((), ())),
        preferred_element_type=jnp.float32)

    @pl.when(i == pl.num_programs(0) - 1)
    def _done():
        cnt = jnp.maximum(cnt_ref[:, 0:1], 1.0)
        hg = acc_ref[...] / cnt
        logit = jnp.dot(hg, pw_ref[...], preferred_element_type=jnp.float32)
        p = logit[:, 0] + pb_ref[0, 0]
        pred_ref[0] = p
        norm_ref[0] = 2.0 * ((p - (-10.0)) / 6.0) - 1.0


def _pool_head(h, batch2d, pred_Wp, pred_b2d):
    bm = 1024
    return pl.pallas_call(
        _pool_body,
        grid=(NP // bm,),
        in_specs=[
            pl.BlockSpec((NC, bm, H), lambda i: (0, i, 0)),
            pl.BlockSpec((1, 1, bm), lambda i: (i, 0, 0)),
            pl.BlockSpec((EP, 128), lambda i: (0, 0)),
            pl.BlockSpec((1, 1), lambda i: (0, 0), memory_space=pltpu.SMEM),
        ],
        out_specs=[
            pl.BlockSpec((1, G), lambda i: (0, 0)),
            pl.BlockSpec((1, G), lambda i: (0, 0)),
        ],
        out_shape=[
            jax.ShapeDtypeStruct((1, G), jnp.float32),
            jax.ShapeDtypeStruct((1, G), jnp.float32),
        ],
        scratch_shapes=[
            pltpu.VMEM((G, EP), jnp.float32),
            pltpu.VMEM((G, 128), jnp.float32),
        ],
    )(h, batch2d, pred_Wp, pred_b2d)


def _sc_layer_body(h_hbm, e_hbm, src_hbm, dst_hbm, out_hbm,
                   acc, src_blk, dst_blk, rows_v, e_v, msg_v,
                   idx_sem, e_sem, gat_sem, sct_sem):
    c = lax.axis_index("c")
    s = lax.axis_index("s")

    def _zrow(i, carry):
        for v in range(H // 16):
            msg_v[0, i, pl.ds(v * 16, 16)] = jnp.zeros((16,), jnp.float32)
        return carry
    lax.fori_loop(0, CH, _zrow, None)
    for j in range(RT // CH):
        pltpu.sync_copy(msg_v.at[0], acc.at[pl.ds(s * RT + j * CH, CH)])

    plsc.subcore_barrier()

    def _issue_blk(bnum):
        slot = lax.rem(bnum, 3)
        pltpu.async_copy(src_hbm.at[s, pl.ds(bnum * BLK, BLK)],
                         src_blk.at[slot], idx_sem.at[slot])
        pltpu.async_copy(dst_hbm.at[s, pl.ds(bnum * BLK, BLK)],
                         dst_blk.at[slot], idx_sem.at[slot])

    def _wait_blk(bnum):
        slot = lax.rem(bnum, 3)
        pltpu.make_async_copy(src_hbm.at[s, pl.ds(0, BLK)],
                              src_blk.at[slot], idx_sem.at[slot]).wait()
        pltpu.make_async_copy(dst_hbm.at[s, pl.ds(0, BLK)],
                              dst_blk.at[slot], idx_sem.at[slot]).wait()

    def _issue_e(k):
        b = lax.rem(k, 2)
        pltpu.async_copy(e_hbm.at[c, pl.ds(s * EW + k * CH, CH)],
                         e_v.at[b], e_sem.at[b])

    def _issue_gather(k):
        b = lax.rem(k, 2)
        slot = lax.rem(k // BLK, 3)
        g = lax.rem(k, BLK)
        pltpu.async_copy(h_hbm.at[c].at[src_blk.at[slot, g]],
                         rows_v.at[b], gat_sem.at[b])

    def _wait_scatter(b):
        pltpu.make_async_copy(msg_v.at[b], acc.at[dst_blk.at[0, 0]],
                              sct_sem.at[b]).wait()

    _issue_blk(0)
    _issue_blk(1)
    _wait_blk(0)
    _issue_e(0)
    _issue_gather(0)

    def _chunk(k, carry):
        b = lax.rem(k, 2)
        slot = lax.rem(k // BLK, 3)
        g = lax.rem(k, BLK)

        pltpu.make_async_copy(h_hbm.at[c].at[src_blk.at[slot, g]],
                              rows_v.at[b], gat_sem.at[b]).wait()
        pltpu.make_async_copy(e_hbm.at[c, pl.ds(0, CH)], e_v.at[b],
                              e_sem.at[b]).wait()

        @pl.when(k >= 2)
        def _():
            _wait_scatter(b)

        @pl.when(lax.rem(k + 1, BLK) == 0)
        def _():
            bnext = (k + 1) // BLK

            @pl.when(bnext < NB)
            def _():
                _wait_blk(bnext)

            @pl.when(bnext + 1 < NB)
            def _():
                _issue_blk(bnext + 1)

        @pl.when(k + 1 < NK)
        def _():
            _issue_gather(k + 1)
            _issue_e(k + 1)

        @plsc.parallel_loop(0, CH, 1, unroll=8)
        def _edge(j):
            for v in range(H // 32):
                sl = pl.ds(v * 16, 16)
                rw = rows_v[b, j, sl]
                ew = e_v[b, j, sl]
                ra = jax.lax.bitcast_convert_type(rw << 16, jnp.float32)
                rb = jax.lax.bitcast_convert_type(rw & -65536, jnp.float32)
                ea = jax.lax.bitcast_convert_type(ew << 16, jnp.float32)
                eb = jax.lax.bitcast_convert_type(ew & -65536, jnp.float32)
                msg_v[b, j, pl.ds(v * 32, 16)] = jnp.maximum(ra + ea, 0.0)
                msg_v[b, j, pl.ds(v * 32 + 16, 16)] = jnp.maximum(rb + eb, 0.0)

        pltpu.async_copy(msg_v.at[b], acc.at[dst_blk.at[slot, g]],
                         sct_sem.at[b], add=True)
        return carry
    lax.fori_loop(0, NK, _chunk, None)

    _wait_scatter(lax.rem(NK - 2, 2))
    _wait_scatter(lax.rem(NK - 1, 2))

    plsc.subcore_barrier()

    pltpu.sync_copy(acc.at[pl.ds(s * RT, RT)],
                    out_hbm.at[c].at[pl.ds(s * RT, RT)])


def _sc_layer(h_bf, e_bf, src_t, dst_t):
    mesh = plsc.VectorSubcoreMesh(core_axis_name="c", subcore_axis_name="s",
                                  num_cores=NC, num_subcores=NS)
    f = pl.kernel(
        _sc_layer_body,
        out_type=jax.ShapeDtypeStruct((NC, NP, H), jnp.float32),
        mesh=mesh,
        compiler_params=pltpu.CompilerParams(use_tc_tiling_on_sc=False, needs_layout_passes=False),
        scratch_types=[
            pltpu.VMEM_SHARED((NP, H), jnp.float32),
            pltpu.VMEM((3, BLK, CH), jnp.int32),
            pltpu.VMEM((3, BLK, CH), jnp.int32),
            pltpu.VMEM((2, CH, HW // 2), jnp.int32),
            pltpu.VMEM((2, CH, HW // 2), jnp.int32),
            pltpu.VMEM((2, CH, H), jnp.float32),
            pltpu.SemaphoreType.DMA((3,)),
            pltpu.SemaphoreType.DMA((2,)),
            pltpu.SemaphoreType.DMA((2,)),
            pltpu.SemaphoreType.DMA((2,)),
        ],
    )
    return f(h_bf, e_bf, src_t, dst_t)


def kernel(x, edge_index, edge_attr, batch, node_W, node_b, edge_W, edge_b,
           W1, b1, W2, b2, eps, pred_W, pred_b):
    f32 = jnp.float32
    node_Wp = jnp.zeros((128, EP), f32).at[:, :300].set(node_W)
    node_bp = jnp.zeros((1, EP), f32).at[0, :300].set(node_b)
    edge_Wp = jnp.zeros((NL, 16, EP), f32).at[:, :, :300].set(edge_W)
    edge_bp = jnp.zeros((NL, 1, EP), f32).at[:, 0, :300].set(edge_b)
    W1p = jnp.zeros((NL, EP, 640), f32).at[:, :300, :600].set(W1)
    b1p = jnp.zeros((NL, 1, 640), f32).at[:, 0, :600].set(b1)
    W2p = jnp.zeros((NL, 640, EP), f32).at[:, :600, :300].set(W2)
    b2p = jnp.zeros((NL, 1, EP), f32).at[:, 0, :300].set(b2)
    pred_Wp = jnp.zeros((EP, 128), f32).at[:300, 0].set(pred_W[:, 0])
    pred_b2d = pred_b.reshape(1, 1)
    eps2d = eps.reshape(NL, 1, 1)

    src_t = edge_index[0].reshape(NS, NK, CH)
    dst_t = edge_index[1].reshape(NS, NK, CH)
    x_p = jnp.zeros((NP, 128), f32).at[:N].set(x)
    batch_p = jnp.full((NP,), -1, jnp.int32).at[:N].set(batch)
    batch2d = batch_p.reshape(NP // 1024, 1, 1024)

    h, h_bf = _encode(x_p, node_Wp, node_bp)
    e_bfs = [_edge_mlp(edge_attr, edge_Wp[l], edge_bp[l]) for l in range(NL)]
    for l in range(NL):
        agg = _sc_layer(h_bf, e_bfs[l], src_t, dst_t)
        h, h_bf = _gin_mlp(h, agg, W1p[l], b1p[l], W2p[l], b2p[l], eps2d[l],
                           last=(l == NL - 1))

    pred2d, norm2d = _pool_head(h, batch2d, pred_Wp, pred_b2d)
    return (pred2d[0], norm2d[0])


# speedup vs baseline: 1.5625x; 1.1389x over previous
import functools

import jax
import jax.numpy as jnp
from jax import lax
from jax.experimental import pallas as pl
from jax.experimental.pallas import tpu as pltpu
from jax.experimental.pallas import tpu_sc as plsc

N = 10000
NP = 10240
E = 320000
G = 64
EP = 320
H = EP // 2
NL = 5

NC, NS = 2, 16
EW = E // NS
CH = 40
NK = EW // CH
BLK = 10
NB = NK // BLK
RT = NP // NS

HW = EP // 2


def _pack_words(r):
    def bits(x):
        return jax.lax.bitcast_convert_type(
            x.astype(jnp.bfloat16), jnp.int16).astype(jnp.int32)
    w0 = (bits(r[:, 0:80]) & 0xFFFF) | (bits(r[:, 80:160]) << 16)
    w1 = (bits(r[:, 160:240]) & 0xFFFF) | (bits(r[:, 240:320]) << 16)
    return jnp.concatenate([w0, w1], axis=1)


def _encode_body(x_ref, w_ref, b_ref, out_ref, outb_ref):
    r = jnp.dot(x_ref[...], w_ref[...], preferred_element_type=jnp.float32)
    r = r + b_ref[...]
    out_ref[0] = r[:, :H]
    out_ref[1] = r[:, H:]
    ri = _pack_words(r)
    outb_ref[0] = ri[:, :HW // 2]
    outb_ref[1] = ri[:, HW // 2:]


def _encode(x, node_Wp, node_bp):
    bm = 1024
    return pl.pallas_call(
        _encode_body,
        grid=(NP // bm,),
        in_specs=[
            pl.BlockSpec((bm, 128), lambda i: (i, 0)),
            pl.BlockSpec((128, EP), lambda i: (0, 0)),
            pl.BlockSpec((1, EP), lambda i: (0, 0)),
        ],
        out_specs=[
            pl.BlockSpec((NC, bm, H), lambda i: (0, i, 0)),
            pl.BlockSpec((NC, bm, HW // 2), lambda i: (0, i, 0)),
        ],
        out_shape=[
            jax.ShapeDtypeStruct((NC, NP, H), jnp.float32),
            jax.ShapeDtypeStruct((NC, NP, HW // 2), jnp.int32),
        ],
    )(x, node_Wp, node_bp)


def _edge_mlp_body(a_ref, w_ref, b_ref, out_ref):
    a16 = a_ref[...].astype(jnp.bfloat16)
    w16 = w_ref[...].astype(jnp.bfloat16)
    r = jnp.dot(a16, w16, preferred_element_type=jnp.float32)
    r = jnp.maximum(r + b_ref[...], 0.0)
    ri = _pack_words(r)
    out_ref[0] = ri[:, :HW // 2]
    out_ref[1] = ri[:, HW // 2:]


def _edge_mlp(edge_attr, Wl, bl):
    bm = 2000
    return pl.pallas_call(
        _edge_mlp_body,
        grid=(E // bm,),
        in_specs=[
            pl.BlockSpec((bm, 16), lambda i: (i, 0)),
            pl.BlockSpec((16, EP), lambda i: (0, 0)),
            pl.BlockSpec((1, EP), lambda i: (0, 0)),
        ],
        out_specs=pl.BlockSpec((NC, bm, HW // 2), lambda i: (0, i, 0)),
        out_shape=jax.ShapeDtypeStruct((NC, E, HW // 2), jnp.int32),
    )(edge_attr, Wl, bl)


def _mlp_body(last, h_ref, agg_ref, w1_ref, b1_ref, w2_ref, b2_ref, eps_ref,
              out_ref, outb_ref):
    hb = jnp.concatenate([h_ref[0], h_ref[1]], axis=1)
    ab = jnp.concatenate([agg_ref[0], agg_ref[1]], axis=1)
    h2 = (1.0 + eps_ref[0, 0]) * hb + ab
    t = jnp.maximum(
        jnp.dot(h2, w1_ref[...], preferred_element_type=jnp.float32)
        + b1_ref[...], 0.0)
    o = jnp.dot(t, w2_ref[...], preferred_element_type=jnp.float32) + b2_ref[...]
    if not last:
        o = jnp.maximum(o, 0.0)
    out_ref[0] = o[:, :H]
    out_ref[1] = o[:, H:]
    oi = _pack_words(o)
    outb_ref[0] = oi[:, :HW // 2]
    outb_ref[1] = oi[:, HW // 2:]


def _gin_mlp(h, agg, W1l, b1l, W2l, b2l, epsl, last):
    bm = 1024
    return pl.pallas_call(
        functools.partial(_mlp_body, last),
        grid=(NP // bm,),
        in_specs=[
            pl.BlockSpec((NC, bm, H), lambda i: (0, i, 0)),
            pl.BlockSpec((NC, bm, H), lambda i: (0, i, 0)),
            pl.BlockSpec((EP, 640), lambda i: (0, 0)),
            pl.BlockSpec((1, 640), lambda i: (0, 0)),
            pl.BlockSpec((640, EP), lambda i: (0, 0)),
            pl.BlockSpec((1, EP), lambda i: (0, 0)),
            pl.BlockSpec((1, 1), lambda i: (0, 0), memory_space=pltpu.SMEM),
        ],
        out_specs=[
            pl.BlockSpec((NC, bm, H), lambda i: (0, i, 0)),
            pl.BlockSpec((NC, bm, HW // 2), lambda i: (0, i, 0)),
        ],
        out_shape=[
            jax.ShapeDtypeStruct((NC, NP, H), jnp.float32),
            jax.ShapeDtypeStruct((NC, NP, HW // 2), jnp.int32),
        ],
    )(h, agg, W1l, b1l, W2l, b2l, epsl)


def _pool_body(h_ref, batch_ref, pw_ref, pb_ref, pred_ref, norm_ref,
               acc_ref, cnt_ref):
    i = pl.program_id(0)

    @pl.when(i == 0)
    def _init():
        acc_ref[...] = jnp.zeros_like(acc_ref)
        cnt_ref[...] = jnp.zeros_like(cnt_ref)

    hb = jnp.concatenate([h_ref[0], h_ref[1]], axis=1)
    bm = hb.shape[0]
    gids = jax.lax.broadcasted_iota(jnp.int32, (bm, G), 1)
    onehot = (batch_ref[0, 0][:, None] == gids).astype(jnp.float32)
    acc_ref[...] += jax.lax.dot_general(
        onehot, hb, (((0,), (0,)), ((), ())),
        preferred_element_type=jnp.float32)
    cnt_ref[...] += jax.lax.dot_general(
        onehot, jnp.ones((bm, 128), jnp.float32), (((0,), (0,)), ((), ())),
        preferred_element_type=jnp.float32)

    @pl.when(i == pl.num_programs(0) - 1)
    def _done():
        cnt = jnp.maximum(cnt_ref[:, 0:1], 1.0)
        hg = acc_ref[...] / cnt
        logit = jnp.dot(hg, pw_ref[...], preferred_element_type=jnp.float32)
        p = logit[:, 0] + pb_ref[0, 0]
        pred_ref[0] = p
        norm_ref[0] = 2.0 * ((p - (-10.0)) / 6.0) - 1.0


def _pool_head(h, batch2d, pred_Wp, pred_b2d):
    bm = 1024
    return pl.pallas_call(
        _pool_body,
        grid=(NP // bm,),
        in_specs=[
            pl.BlockSpec((NC, bm, H), lambda i: (0, i, 0)),
            pl.BlockSpec((1, 1, bm), lambda i: (i, 0, 0)),
            pl.BlockSpec((EP, 128), lambda i: (0, 0)),
            pl.BlockSpec((1, 1), lambda i: (0, 0), memory_space=pltpu.SMEM),
        ],
        out_specs=[
            pl.BlockSpec((1, G), lambda i: (0, 0)),
            pl.BlockSpec((1, G), lambda i: (0, 0)),
        ],
        out_shape=[
            jax.ShapeDtypeStruct((1, G), jnp.float32),
            jax.ShapeDtypeStruct((1, G), jnp.float32),
        ],
        scratch_shapes=[
            pltpu.VMEM((G, EP), jnp.float32),
            pltpu.VMEM((G, 128), jnp.float32),
        ],
    )(h, batch2d, pred_Wp, pred_b2d)


def _sc_layer_body(h_hbm, e_hbm, src_hbm, dst_hbm, out_hbm,
                   acc, src_blk, dst_blk, rows_v, e_v, msg_v,
                   idx_sem, e_sem, gat_sem, sct_sem):
    c = lax.axis_index("c")
    s = lax.axis_index("s")

    def _zrow(i, carry):
        for v in range(H // 16):
            msg_v[0, i, pl.ds(v * 16, 16)] = jnp.zeros((16,), jnp.float32)
        return carry
    lax.fori_loop(0, CH, _zrow, None)
    for j in range(RT // CH):
        pltpu.sync_copy(msg_v.at[0], acc.at[pl.ds(s * RT + j * CH, CH)])

    plsc.subcore_barrier()

    def _issue_blk(bnum):
        slot = lax.rem(bnum, 3)
        pltpu.async_copy(src_hbm.at[s, pl.ds(bnum * BLK, BLK)],
                         src_blk.at[slot], idx_sem.at[slot])
        pltpu.async_copy(dst_hbm.at[s, pl.ds(bnum * BLK, BLK)],
                         dst_blk.at[slot], idx_sem.at[slot])

    def _wait_blk(bnum):
        slot = lax.rem(bnum, 3)
        pltpu.make_async_copy(src_hbm.at[s, pl.ds(0, BLK)],
                              src_blk.at[slot], idx_sem.at[slot]).wait()
        pltpu.make_async_copy(dst_hbm.at[s, pl.ds(0, BLK)],
                              dst_blk.at[slot], idx_sem.at[slot]).wait()

    def _issue_e(k):
        b = lax.rem(k, 2)
        pltpu.async_copy(e_hbm.at[c, pl.ds(s * EW + k * CH, CH)],
                         e_v.at[b], e_sem.at[b])

    def _issue_gather(k):
        b = lax.rem(k, 2)
        slot = lax.rem(k // BLK, 3)
        g = lax.rem(k, BLK)
        pltpu.async_copy(h_hbm.at[c].at[src_blk.at[slot, g]],
                         rows_v.at[b], gat_sem.at[b])

    def _wait_scatter(b):
        pltpu.make_async_copy(msg_v.at[b], acc.at[dst_blk.at[0, 0]],
                              sct_sem.at[b]).wait()

    _issue_blk(0)
    _issue_blk(1)
    _wait_blk(0)
    _issue_e(0)
    _issue_gather(0)

    def _chunk(k, carry):
        b = lax.rem(k, 2)
        slot = lax.rem(k // BLK, 3)
        g = lax.rem(k, BLK)

        pltpu.make_async_copy(h_hbm.at[c].at[src_blk.at[slot, g]],
                              rows_v.at[b], gat_sem.at[b]).wait()
        pltpu.make_async_copy(e_hbm.at[c, pl.ds(0, CH)], e_v.at[b],
                              e_sem.at[b]).wait()

        @pl.when(k >= 2)
        def _():
            _wait_scatter(b)

        @pl.when(lax.rem(k + 1, BLK) == 0)
        def _():
            bnext = (k + 1) // BLK

            @pl.when(bnext < NB)
            def _():
                _wait_blk(bnext)

            @pl.when(bnext + 1 < NB)
            def _():
                _issue_blk(bnext + 1)

        @pl.when(k + 1 < NK)
        def _():
            _issue_gather(k + 1)
            _issue_e(k + 1)

        @plsc.parallel_loop(0, CH, 1, unroll=8)
        def _edge(j):
            for v in range(H // 32):
                sl = pl.ds(v * 16, 16)
                rw = rows_v[b, j, sl]
                ew = e_v[b, j, sl]
                ra = jax.lax.bitcast_convert_type(rw << 16, jnp.float32)
                rb = jax.lax.bitcast_convert_type(rw & -65536, jnp.float32)
                ea = jax.lax.bitcast_convert_type(ew << 16, jnp.float32)
                eb = jax.lax.bitcast_convert_type(ew & -65536, jnp.float32)
                msg_v[b, j, pl.ds(v * 16, 16)] = jnp.maximum(ra + ea, 0.0)
                msg_v[b, j, pl.ds(80 + v * 16, 16)] = jnp.maximum(rb + eb, 0.0)

        pltpu.async_copy(msg_v.at[b], acc.at[dst_blk.at[slot, g]],
                         sct_sem.at[b], add=True)
        return carry
    lax.fori_loop(0, NK, _chunk, None)

    _wait_scatter(lax.rem(NK - 2, 2))
    _wait_scatter(lax.rem(NK - 1, 2))

    plsc.subcore_barrier()

    pltpu.sync_copy(acc.at[pl.ds(s * RT, RT)],
                    out_hbm.at[c].at[pl.ds(s * RT, RT)])


def _sc_layer(h_bf, e_bf, src_t, dst_t):
    mesh = plsc.VectorSubcoreMesh(core_axis_name="c", subcore_axis_name="s",
                                  num_cores=NC, num_subcores=NS)
    f = pl.kernel(
        _sc_layer_body,
        out_type=jax.ShapeDtypeStruct((NC, NP, H), jnp.float32),
        mesh=mesh,
        compiler_params=pltpu.CompilerParams(use_tc_tiling_on_sc=False, needs_layout_passes=False),
        scratch_types=[
            pltpu.VMEM_SHARED((NP, H), jnp.float32),
            pltpu.VMEM((3, BLK, CH), jnp.int32),
            pltpu.VMEM((3, BLK, CH), jnp.int32),
            pltpu.VMEM((2, CH, HW // 2), jnp.int32),
            pltpu.VMEM((2, CH, HW // 2), jnp.int32),
            pltpu.VMEM((2, CH, H), jnp.float32),
            pltpu.SemaphoreType.DMA((3,)),
            pltpu.SemaphoreType.DMA((2,)),
            pltpu.SemaphoreType.DMA((2,)),
            pltpu.SemaphoreType.DMA((2,)),
        ],
    )
    return f(h_bf, e_bf, src_t, dst_t)


def kernel(x, edge_index, edge_attr, batch, node_W, node_b, edge_W, edge_b,
           W1, b1, W2, b2, eps, pred_W, pred_b):
    f32 = jnp.float32
    node_Wp = jnp.zeros((128, EP), f32).at[:, :300].set(node_W)
    node_bp = jnp.zeros((1, EP), f32).at[0, :300].set(node_b)
    edge_Wp = jnp.zeros((NL, 16, EP), f32).at[:, :, :300].set(edge_W)
    edge_bp = jnp.zeros((NL, 1, EP), f32).at[:, 0, :300].set(edge_b)
    W1p = jnp.zeros((NL, EP, 640), f32).at[:, :300, :600].set(W1)
    b1p = jnp.zeros((NL, 1, 640), f32).at[:, 0, :600].set(b1)
    W2p = jnp.zeros((NL, 640, EP), f32).at[:, :600, :300].set(W2)
    b2p = jnp.zeros((NL, 1, EP), f32).at[:, 0, :300].set(b2)
    pred_Wp = jnp.zeros((EP, 128), f32).at[:300, 0].set(pred_W[:, 0])
    pred_b2d = pred_b.reshape(1, 1)
    eps2d = eps.reshape(NL, 1, 1)

    src_t = edge_index[0].reshape(NS, NK, CH)
    dst_t = edge_index[1].reshape(NS, NK, CH)
    x_p = jnp.zeros((NP, 128), f32).at[:N].set(x)
    batch_p = jnp.full((NP,), -1, jnp.int32).at[:N].set(batch)
    batch2d = batch_p.reshape(NP // 1024, 1, 1024)

    h, h_bf = _encode(x_p, node_Wp, node_bp)
    e_bfs = [_edge_mlp(edge_attr, edge_Wp[l], edge_bp[l]) for l in range(NL)]
    for l in range(NL):
        agg = _sc_layer(h_bf, e_bfs[l], src_t, dst_t)
        h, h_bf = _gin_mlp(h, agg, W1p[l], b1p[l], W2p[l], b2p[l], eps2d[l],
                           last=(l == NL - 1))

    pred2d, norm2d = _pool_head(h, batch2d, pred_Wp, pred_b2d)
    return (pred2d[0], norm2d[0])
